# baseline (speedup 1.0000x reference)
import functools

import jax
import jax.numpy as jnp
from jax import lax
from jax.experimental import pallas as pl
from jax.experimental.pallas import tpu as pltpu
from jax.experimental.pallas import tpu_sc as plsc

_B, _T, _H, _V = 1024, 200, 64, 100000
_NST, _NKC, _NKR, _NOBS = 3, 2, 4, 3
_NW = 32
_VB = 128
_VP = 100096
_NBLK = _VP // _VB
_K1_ITERS = -(-_NBLK // _NW)
_NUNIT = _T * 8
_UPW = _NUNIT // _NW

_MESH = plsc.VectorSubcoreMesh(core_axis_name="c", subcore_axis_name="s")
_CPARAMS = pltpu.CompilerParams(needs_layout_passes=False)


@functools.partial(
    pl.kernel,
    out_type=(
        jax.ShapeDtypeStruct((_VP, 2 * _H), jnp.float32),
        jax.ShapeDtypeStruct((_VP, 2 * _H), jnp.float32),
        jax.ShapeDtypeStruct((_VP, 2 * _H), jnp.float32),
    ),
    mesh=_MESH,
    compiler_params=_CPARAMS,
    scratch_types=[
        pltpu.VMEM((5, _H, _VB), jnp.float32),
        pltpu.VMEM((_VB, 2 * _H), jnp.float32),
        pltpu.VMEM((_VB, 2 * _H), jnp.float32),
        pltpu.VMEM((_VB, 2 * _H), jnp.float32),
        pltpu.SemaphoreType.DMA,
        pltpu.SemaphoreType.DMA,
        pltpu.SemaphoreType.DMA,
        pltpu.SemaphoreType.DMA,
        pltpu.SemaphoreType.DMA,
        pltpu.SemaphoreType.DMA,
    ],
)
def _sc_reformat(ekcv, estv, ekc2, esta, estb,
                 tin, tkc, tsta, tstb, si0, si1, si2, si3, si4, sem_out):
    wid = lax.axis_index("s") * 2 + lax.axis_index("c")
    lane = lax.iota(jnp.int32, 16)
    rowv = [lane + c * 16 for c in range(_VB // 16)]
    sin = [si0, si1, si2, si3, si4]

    def transpose_field(slot, dst, coff):
        pltpu.make_async_copy(ekcv.at[0, :, pl.ds(0, _VB)],
                              tin.at[slot], sin[slot]).wait()

        def hbody(h, c2):
            col = jnp.full((16,), coff + h, jnp.int32)
            for c in range(_VB // 16):
                val = tin[slot, h, pl.ds(c * 16, 16)]
                plsc.store_scatter(dst, [rowv[c], col], val)
            return c2

        lax.fori_loop(0, _H, hbody, 0)

    def block(it, carry):
        blk = it * _NW + wid

        @pl.when(blk < _NBLK)
        def _():
            v0 = pl.multiple_of(blk * _VB, _VB)
            for f in range(_NKC):
                pltpu.async_copy(ekcv.at[f, :, pl.ds(v0, _VB)],
                                 tin.at[f], sin[f])
            for f in range(_NST):
                pltpu.async_copy(estv.at[f, :, pl.ds(v0, _VB)],
                                 tin.at[_NKC + f], sin[_NKC + f])

            @pl.when(it > 0)
            def _():
                pltpu.make_async_copy(tkc, ekc2.at[pl.ds(0, _VB)],
                                      sem_out).wait()
                pltpu.make_async_copy(tsta, esta.at[pl.ds(0, _VB)],
                                      sem_out).wait()
                pltpu.make_async_copy(tstb, estb.at[pl.ds(0, _VB)],
                                      sem_out).wait()

            transpose_field(0, tkc, 0)
            transpose_field(1, tkc, _H)
            pltpu.async_copy(tkc, ekc2.at[pl.ds(v0, _VB)], sem_out)
            transpose_field(2, tsta, 0)
            transpose_field(3, tsta, _H)
            pltpu.async_copy(tsta, esta.at[pl.ds(v0, _VB)], sem_out)
            transpose_field(4, tstb, 0)
            pltpu.async_copy(tstb, estb.at[pl.ds(v0, _VB)], sem_out)

        return carry

    lax.fori_loop(0, _K1_ITERS, block, 0)
    pltpu.make_async_copy(tkc, ekc2.at[pl.ds(0, _VB)], sem_out).wait()
    pltpu.make_async_copy(tsta, esta.at[pl.ds(0, _VB)], sem_out).wait()
    pltpu.make_async_copy(tstb, estb.at[pl.ds(0, _VB)], sem_out).wait()


@functools.partial(
    pl.kernel,
    out_type=(
        jax.ShapeDtypeStruct((_T * 6 * 8, 8, 8, 128), jnp.float32),
        jax.ShapeDtypeStruct((_NST * 8, 8, 8, 128), jnp.float32),
    ),
    mesh=_MESH,
    compiler_params=_CPARAMS,
    scratch_types=[
        pltpu.VMEM((128,), jnp.int32),
        pltpu.VMEM((128,), jnp.int32),
        pltpu.VMEM((128, 2 * _H), jnp.float32),
        pltpu.VMEM((128, 2 * _H), jnp.float32),
        pltpu.VMEM((16, 8, 128), jnp.float32),
        pltpu.VMEM((8, 8, 128), jnp.float32),
        pltpu.SemaphoreType.DMA,
        pltpu.SemaphoreType.DMA,
        pltpu.SemaphoreType.DMA,
    ],
)
def _sc_gather(kcidx, statx, ekc2, esta, estb,
               ko, so,
               i0, i1, ga, gb, kslab, sslab, sem_ii, sem_g, sem_ko):
    wid = lax.axis_index("s") * 2 + lax.axis_index("c")
    lane = lax.iota(jnp.int32, 16)
    rowbase = [lane + c * 16 for c in range(8)]

    def fire_in(u):
        t = u // 8
        b0 = pl.multiple_of((u % 8) * 128, 128)
        pltpu.async_copy(kcidx.at[t, 0, pl.ds(b0, 128)], i0, sem_ii)
        pltpu.async_copy(kcidx.at[t, 1, pl.ds(b0, 128)], i1, sem_ii)

    def kc_slab(slab, g, coff, row0):
        def hbody(h, c2):
            hq = h // 8
            hr = h % 8
            colv = jnp.full((16,), coff + h, jnp.int32)
            for c in range(8):
                val = plsc.load_gather(g, [rowbase[c], colv])
                slab[row0 + hq, hr, pl.ds(c * 16, 16)] = val
            return c2

        lax.fori_loop(0, _H, hbody, 0)

    fire_in(wid * _UPW)

    def unit(ui, carry):
        u = wid * _UPW + ui
        t = u // 8
        bb = u % 8
        pltpu.make_async_copy(kcidx.at[0, 0, pl.ds(0, 128)], i0, sem_ii).wait()
        pltpu.make_async_copy(kcidx.at[0, 1, pl.ds(0, 128)], i1, sem_ii).wait()
        pltpu.async_copy(ekc2.at[i0], ga, sem_g)
        pltpu.async_copy(ekc2.at[i1], gb, sem_g)

        @pl.when(ui > 0)
        def _():
            pltpu.make_async_copy(kslab, ko.at[pl.ds(0, 16), 0], sem_ko).wait()

        pltpu.make_async_copy(ekc2.at[pl.ds(0, 128)], ga, sem_g).wait()
        pltpu.make_async_copy(ekc2.at[pl.ds(0, 128)], gb, sem_g).wait()

        @pl.when(ui + 1 < _UPW)
        def _():
            fire_in(u + 1)

        kc_slab(kslab, ga, 0, 0)
        kc_slab(kslab, gb, _H, 8)
        pltpu.async_copy(kslab, ko.at[pl.ds(t * 48 + 32, 16), bb], sem_ko)
        return carry

    lax.fori_loop(0, _UPW, unit, 0)
    pltpu.make_async_copy(kslab, ko.at[pl.ds(0, 16), 0], sem_ko).wait()

    @pl.when(wid < _NST * 8)
    def _():
        f = wid // 8
        bb = wid % 8
        b0 = pl.multiple_of(bb * 128, 128)
        pltpu.sync_copy(statx.at[f, pl.ds(b0, 128)], i0)
        for ff in range(2):
            @pl.when(f == ff)
            def _():
                pltpu.async_copy(esta.at[i0], ga, sem_g).wait()

        @pl.when(f == 2)
        def _():
            pltpu.async_copy(estb.at[i0], ga, sem_g).wait()

        coff = (f % 2) * _H

        def hbody(h, c2):
            hq = h // 8
            hr = h % 8
            colv = jnp.full((16,), coff + h, jnp.int32)
            for c in range(8):
                val = plsc.load_gather(ga, [rowbase[c], colv])
                sslab[hq, hr, pl.ds(c * 16, 16)] = val
            return c2

        lax.fori_loop(0, _H, hbody, 0)
        pltpu.sync_copy(sslab, so.at[pl.ds(f * 8, 8), bb])


def _tc_known_body(s_ref, w_ref, b_ref, kin_ref, ko_ref):
    del kin_ref
    j = pl.program_id(1)
    w = w_ref[j, :]
    b = b_ref[j, :]
    s = s_ref[0, j, :]
    ko_ref[0, 0] = w[:, None] * s[None, :] + b[:, None]


def _tc_obs_body(s_ref, w_ref, b_ref, oo_ref):
    j = pl.program_id(1)
    w = w_ref[j, :]
    b = b_ref[j, :]
    s = s_ref[0, _NKR + j, :]
    oo_ref[0, 0] = w[:, None] * s[None, :] + b[:, None]


_tc_known = pl.pallas_call(
    _tc_known_body,
    grid=(_T, _NKR),
    in_specs=[
        pl.BlockSpec((1, 8, _B), lambda t, j: (t, 0, 0)),
        pl.BlockSpec((_NKR, _H), lambda t, j: (0, 0)),
        pl.BlockSpec((_NKR, _H), lambda t, j: (0, 0)),
        pl.BlockSpec((1, 1, _H, _B), lambda t, j: (t, j, 0, 0)),
    ],
    out_specs=pl.BlockSpec((1, 1, _H, _B), lambda t, j: (t, j, 0, 0)),
    out_shape=jax.ShapeDtypeStruct((_T, 6, _H, _B), jnp.float32),
    input_output_aliases={3: 0},
)

_tc_obs = pl.pallas_call(
    _tc_obs_body,
    grid=(_T, _NOBS),
    in_specs=[
        pl.BlockSpec((1, 8, _B), lambda t, j: (t, 0, 0)),
        pl.BlockSpec((_NOBS, _H), lambda t, j: (0, 0)),
        pl.BlockSpec((_NOBS, _H), lambda t, j: (0, 0)),
    ],
    out_specs=pl.BlockSpec((1, 1, _H, _B), lambda t, j: (t, j, 0, 0)),
    out_shape=jax.ShapeDtypeStruct((_T, _NOBS, _H, _B), jnp.float32),
)


def kernel(static, known_real, known_categorical, observed,
           E_static, E_kc, W_kr, b_kr, W_obs, b_obs):
    ekcv = jnp.swapaxes(E_kc, 1, 2)
    estv = jnp.swapaxes(E_static, 1, 2)
    ekc2, esta, estb = _sc_reformat(ekcv, estv)
    scal2 = jnp.transpose(
        jnp.concatenate(
            [known_real, observed, jnp.zeros((_B, _T, 1), jnp.float32)],
            axis=-1),
        (1, 2, 0))
    kcidx = jnp.transpose(known_categorical.astype(jnp.int32), (1, 2, 0))
    statx = jnp.transpose(static.astype(jnp.int32), (1, 0))
    ko, so = _sc_gather(kcidx, statx, ekc2, esta, estb)
    ko4 = _tc_known(scal2, W_kr, b_kr, ko.reshape(_T, 6, _H, _B))
    oo4 = _tc_obs(scal2, W_obs, b_obs)
    known = (ko4.reshape(_T, 6, 8, 8, 8, 128)
             .transpose(3, 5, 0, 2, 4, 1)
             .reshape(_B, _T, _H, 6))
    obs = (oo4.reshape(_T, 3, 8, 8, 8, 128)
           .transpose(3, 5, 0, 2, 4, 1)
           .reshape(_B, _T, _H, 3))
    static_emb = (so.reshape(_NST, 8, 8, 8, 128)
                  .transpose(2, 4, 0, 1, 3)
                  .reshape(_B, _NST, _H))
    return (static_emb, known, obs)


# speedup vs baseline: 1.1916x; 1.1916x over previous
import functools

import jax
import jax.numpy as jnp
from jax import lax
from jax.experimental import pallas as pl
from jax.experimental.pallas import tpu as pltpu
from jax.experimental.pallas import tpu_sc as plsc

_B, _T, _H, _V = 1024, 200, 64, 100000
_NST, _NKC, _NKR, _NOBS = 3, 2, 4, 3
_NW = 32
_VB = 128
_VP = 100096
_NBLK = _VP // _VB
_K1_ITERS = -(-_NBLK // _NW)
_NUNIT = _T * 8
_UPW = _NUNIT // _NW

_MESH = plsc.VectorSubcoreMesh(core_axis_name="c", subcore_axis_name="s")
_CPARAMS = pltpu.CompilerParams(needs_layout_passes=False)


@functools.partial(
    pl.kernel,
    out_type=(
        jax.ShapeDtypeStruct((_VP, 2 * _H), jnp.float32),
        jax.ShapeDtypeStruct((_VP, 2 * _H), jnp.float32),
        jax.ShapeDtypeStruct((_VP, 2 * _H), jnp.float32),
    ),
    mesh=_MESH,
    compiler_params=_CPARAMS,
    scratch_types=[
        pltpu.VMEM((5, _H, _VB), jnp.float32),
        pltpu.VMEM((_VB, 2 * _H), jnp.float32),
        pltpu.VMEM((_VB, 2 * _H), jnp.float32),
        pltpu.VMEM((_VB, 2 * _H), jnp.float32),
        pltpu.SemaphoreType.DMA,
        pltpu.SemaphoreType.DMA,
        pltpu.SemaphoreType.DMA,
        pltpu.SemaphoreType.DMA,
        pltpu.SemaphoreType.DMA,
        pltpu.SemaphoreType.DMA,
    ],
)
def _sc_reformat(ekcv, estv, ekc2, esta, estb,
                 tin, tkc, tsta, tstb, si0, si1, si2, si3, si4, sem_out):
    wid = lax.axis_index("s") * 2 + lax.axis_index("c")
    lane = lax.iota(jnp.int32, 16)
    rowv = [lane + c * 16 for c in range(_VB // 16)]
    sin = [si0, si1, si2, si3, si4]

    def transpose_field(slot, dst, coff):
        pltpu.make_async_copy(ekcv.at[0, :, pl.ds(0, _VB)],
                              tin.at[slot], sin[slot]).wait()

        def hbody(h, c2):
            col = jnp.full((16,), coff + h, jnp.int32)
            for c in range(_VB // 16):
                val = tin[slot, h, pl.ds(c * 16, 16)]
                plsc.store_scatter(dst, [rowv[c], col], val)
            return c2

        lax.fori_loop(0, _H, hbody, 0)

    def block(it, carry):
        blk = it * _NW + wid

        @pl.when(blk < _NBLK)
        def _():
            v0 = pl.multiple_of(blk * _VB, _VB)
            for f in range(_NKC):
                pltpu.async_copy(ekcv.at[f, :, pl.ds(v0, _VB)],
                                 tin.at[f], sin[f])
            for f in range(_NST):
                pltpu.async_copy(estv.at[f, :, pl.ds(v0, _VB)],
                                 tin.at[_NKC + f], sin[_NKC + f])

            @pl.when(it > 0)
            def _():
                pltpu.make_async_copy(tkc, ekc2.at[pl.ds(0, _VB)],
                                      sem_out).wait()
                pltpu.make_async_copy(tsta, esta.at[pl.ds(0, _VB)],
                                      sem_out).wait()
                pltpu.make_async_copy(tstb, estb.at[pl.ds(0, _VB)],
                                      sem_out).wait()

            transpose_field(0, tkc, 0)
            transpose_field(1, tkc, _H)
            pltpu.async_copy(tkc, ekc2.at[pl.ds(v0, _VB)], sem_out)
            transpose_field(2, tsta, 0)
            transpose_field(3, tsta, _H)
            pltpu.async_copy(tsta, esta.at[pl.ds(v0, _VB)], sem_out)
            transpose_field(4, tstb, 0)
            pltpu.async_copy(tstb, estb.at[pl.ds(v0, _VB)], sem_out)

        return carry

    lax.fori_loop(0, _K1_ITERS, block, 0)
    pltpu.make_async_copy(tkc, ekc2.at[pl.ds(0, _VB)], sem_out).wait()
    pltpu.make_async_copy(tsta, esta.at[pl.ds(0, _VB)], sem_out).wait()
    pltpu.make_async_copy(tstb, estb.at[pl.ds(0, _VB)], sem_out).wait()


@functools.partial(
    pl.kernel,
    out_type=(
        jax.ShapeDtypeStruct((_T * 6 * 8, 8, 8, 128), jnp.float32),
        jax.ShapeDtypeStruct((_NST * 8, 8, 8, 128), jnp.float32),
    ),
    mesh=_MESH,
    compiler_params=_CPARAMS,
    scratch_types=[
        pltpu.VMEM((128,), jnp.int32),
        pltpu.VMEM((128,), jnp.int32),
        pltpu.VMEM((128, 2 * _H), jnp.float32),
        pltpu.VMEM((128, 2 * _H), jnp.float32),
        pltpu.VMEM((16, 8, 128), jnp.float32),
        pltpu.VMEM((8, 8, 128), jnp.float32),
        pltpu.SemaphoreType.DMA,
        pltpu.SemaphoreType.DMA,
        pltpu.SemaphoreType.DMA,
    ],
)
def _sc_gather(kcidx, statx, ekc2, esta, estb,
               ko, so,
               i0, i1, ga, gb, kslab, sslab, sem_ii, sem_g, sem_ko):
    wid = lax.axis_index("s") * 2 + lax.axis_index("c")
    lane = lax.iota(jnp.int32, 16)
    rowbase = [lane + c * 16 for c in range(8)]

    def fire_in(u):
        t = u // 8
        b0 = pl.multiple_of((u % 8) * 128, 128)
        pltpu.async_copy(kcidx.at[t, 0, pl.ds(b0, 128)], i0, sem_ii)
        pltpu.async_copy(kcidx.at[t, 1, pl.ds(b0, 128)], i1, sem_ii)

    def kc_slab(slab, g, coff, row0):
        def hbody(h, c2):
            hq = h // 8
            hr = h % 8
            colv = jnp.full((16,), coff + h, jnp.int32)
            for c in range(8):
                val = plsc.load_gather(g, [rowbase[c], colv])
                slab[row0 + hq, hr, pl.ds(c * 16, 16)] = val
            return c2

        lax.fori_loop(0, _H, hbody, 0)

    fire_in(wid * _UPW)

    def unit(ui, carry):
        u = wid * _UPW + ui
        t = u // 8
        bb = u % 8
        pltpu.make_async_copy(kcidx.at[0, 0, pl.ds(0, 128)], i0, sem_ii).wait()
        pltpu.make_async_copy(kcidx.at[0, 1, pl.ds(0, 128)], i1, sem_ii).wait()
        pltpu.async_copy(ekc2.at[i0], ga, sem_g)
        pltpu.async_copy(ekc2.at[i1], gb, sem_g)

        @pl.when(ui > 0)
        def _():
            pltpu.make_async_copy(kslab, ko.at[pl.ds(0, 16), 0], sem_ko).wait()

        pltpu.make_async_copy(ekc2.at[pl.ds(0, 128)], ga, sem_g).wait()
        pltpu.make_async_copy(ekc2.at[pl.ds(0, 128)], gb, sem_g).wait()

        @pl.when(ui + 1 < _UPW)
        def _():
            fire_in(u + 1)

        kc_slab(kslab, ga, 0, 0)
        kc_slab(kslab, gb, _H, 8)
        pltpu.async_copy(kslab, ko.at[pl.ds(t * 48 + 32, 16), bb], sem_ko)
        return carry

    lax.fori_loop(0, _UPW, unit, 0)
    pltpu.make_async_copy(kslab, ko.at[pl.ds(0, 16), 0], sem_ko).wait()

    @pl.when(wid < _NST * 8)
    def _():
        f = wid // 8
        bb = wid % 8
        b0 = pl.multiple_of(bb * 128, 128)
        pltpu.sync_copy(statx.at[f, pl.ds(b0, 128)], i0)
        for ff in range(2):
            @pl.when(f == ff)
            def _():
                pltpu.async_copy(esta.at[i0], ga, sem_g).wait()

        @pl.when(f == 2)
        def _():
            pltpu.async_copy(estb.at[i0], ga, sem_g).wait()

        coff = (f % 2) * _H

        def hbody(h, c2):
            hq = h // 8
            hr = h % 8
            colv = jnp.full((16,), coff + h, jnp.int32)
            for c in range(8):
                val = plsc.load_gather(ga, [rowbase[c], colv])
                sslab[hq, hr, pl.ds(c * 16, 16)] = val
            return c2

        lax.fori_loop(0, _H, hbody, 0)
        pltpu.sync_copy(sslab, so.at[pl.ds(f * 8, 8), bb])


def _tc_known_body(s_ref, w_ref, b_ref, kin_ref, ko_ref):
    j = pl.program_id(1)

    @pl.when(j < _NKR)
    def _():
        wr = w_ref[0]
        br = b_ref[0]
        sr = s_ref[0, 0]
        ko_ref[0] = (wr[:, None, :, None] * sr[None, :, None, :]
                     + br[:, None, :, None])

    @pl.when(j >= _NKR)
    def _():
        ko_ref[0] = kin_ref[0]


def _tc_obs_body(s_ref, w_ref, b_ref, oo_ref):
    wr = w_ref[0]
    br = b_ref[0]
    sr = s_ref[0, 0]
    oo_ref[0] = (wr[:, None, :, None] * sr[None, :, None, :]
                 + br[:, None, :, None])


_tc_known = pl.pallas_call(
    _tc_known_body,
    grid=(_T, 6),
    in_specs=[
        pl.BlockSpec((1, 1, 8, 128),
                     lambda t, j: (t, jnp.minimum(j, _NKR - 1), 0, 0)),
        pl.BlockSpec((1, 8, 8), lambda t, j: (jnp.minimum(j, _NKR - 1), 0, 0)),
        pl.BlockSpec((1, 8, 8), lambda t, j: (jnp.minimum(j, _NKR - 1), 0, 0)),
        pl.BlockSpec((1, 8, 8, 8, 128),
                     lambda t, j: (t * 6 + jnp.maximum(j, _NKR), 0, 0, 0, 0)),
    ],
    out_specs=pl.BlockSpec((1, 8, 8, 8, 128),
                           lambda t, j: (t * 6 + j, 0, 0, 0, 0)),
    out_shape=jax.ShapeDtypeStruct((_T * 6, 8, 8, 8, 128), jnp.float32),
    input_output_aliases={3: 0},
)

_tc_obs = pl.pallas_call(
    _tc_obs_body,
    grid=(_T, _NOBS),
    in_specs=[
        pl.BlockSpec((1, 1, 8, 128), lambda t, j: (t, _NKR + j, 0, 0)),
        pl.BlockSpec((1, 8, 8), lambda t, j: (j, 0, 0)),
        pl.BlockSpec((1, 8, 8), lambda t, j: (j, 0, 0)),
    ],
    out_specs=pl.BlockSpec((1, 8, 8, 8, 128),
                           lambda t, j: (t * 3 + j, 0, 0, 0, 0)),
    out_shape=jax.ShapeDtypeStruct((_T * 3, 8, 8, 8, 128), jnp.float32),
)


def kernel(static, known_real, known_categorical, observed,
           E_static, E_kc, W_kr, b_kr, W_obs, b_obs):
    ekcv = jnp.swapaxes(E_kc, 1, 2)
    estv = jnp.swapaxes(E_static, 1, 2)
    ekc2, esta, estb = _sc_reformat(ekcv, estv)
    scal2 = jnp.transpose(
        jnp.concatenate(
            [known_real, observed, jnp.zeros((_B, _T, 1), jnp.float32)],
            axis=-1),
        (1, 2, 0))
    kcidx = jnp.transpose(known_categorical.astype(jnp.int32), (1, 2, 0))
    statx = jnp.transpose(static.astype(jnp.int32), (1, 0))
    ko, so = _sc_gather(kcidx, statx, ekc2, esta, estb)
    scal6 = scal2.reshape(_T, 8, 8, 128)
    ko5 = _tc_known(scal6, W_kr.reshape(_NKR, 8, 8), b_kr.reshape(_NKR, 8, 8),
                    ko.reshape(_T * 6, 8, 8, 8, 128))
    oo5 = _tc_obs(scal6, W_obs.reshape(_NOBS, 8, 8), b_obs.reshape(_NOBS, 8, 8))
    known = (ko5.reshape(_T, 6, 8, 8, 8, 128)
             .transpose(3, 5, 0, 2, 4, 1)
             .reshape(_B, _T, _H, 6))
    obs = (oo5.reshape(_T, 3, 8, 8, 8, 128)
           .transpose(3, 5, 0, 2, 4, 1)
           .reshape(_B, _T, _H, 3))
    static_emb = (so.reshape(_NST, 8, 8, 8, 128)
                  .transpose(2, 4, 0, 1, 3)
                  .reshape(_B, _NST, _H))
    return (static_emb, known, obs)


# speedup vs baseline: 1.4503x; 1.2171x over previous
import functools

import jax
import jax.numpy as jnp
from jax import lax
from jax.experimental import pallas as pl
from jax.experimental.pallas import tpu as pltpu
from jax.experimental.pallas import tpu_sc as plsc

_B, _T, _H, _V = 1024, 200, 64, 100000
_NST, _NKC, _NKR, _NOBS = 3, 2, 4, 3
_NW = 32
_VB = 256
_VP = 100096
_NBLK = _VP // _VB
_K1_ITERS = -(-_NBLK // _NW)
_NUNIT = _T * 8
_UPW = _NUNIT // _NW

_MESH = plsc.VectorSubcoreMesh(core_axis_name="c", subcore_axis_name="s")
_CPARAMS = pltpu.CompilerParams(needs_layout_passes=False)


@functools.partial(
    pl.kernel,
    out_type=(
        jax.ShapeDtypeStruct((_VP, 2 * _H), jnp.float32),
        jax.ShapeDtypeStruct((_VP, 2 * _H), jnp.float32),
        jax.ShapeDtypeStruct((_VP, 2 * _H), jnp.float32),
    ),
    mesh=_MESH,
    compiler_params=_CPARAMS,
    scratch_types=[
        pltpu.VMEM((_H, _VB), jnp.float32),
        pltpu.VMEM((_VB, 2 * _H), jnp.float32),
        pltpu.VMEM((_VB, 2 * _H), jnp.float32),
        pltpu.VMEM((_VB, 2 * _H), jnp.float32),
        pltpu.SemaphoreType.DMA,
    ],
)
def _sc_reformat(ekcv, estv, ekc2, esta, estb,
                 tin, tkc, tsta, tstb, sem_out):
    wid = lax.axis_index("s") * 2 + lax.axis_index("c")
    lane = lax.iota(jnp.int32, 16)
    rowv = [lane + c * 16 for c in range(_VB // 16)]

    def transpose_field(src, f, v0, dst, coff):
        pltpu.sync_copy(src.at[f, :, pl.ds(v0, _VB)], tin)

        def hbody(h, c2):
            col = jnp.full((16,), coff + h, jnp.int32)
            for c in range(_VB // 16):
                val = tin[h, pl.ds(c * 16, 16)]
                plsc.store_scatter(dst, [rowv[c], col], val)
            return c2

        lax.fori_loop(0, _H, hbody, 0)

    def block(it, carry):
        blk = it * _NW + wid

        @pl.when(blk < _NBLK)
        def _():
            v0 = pl.multiple_of(blk * _VB, _VB)

            @pl.when(it > 0)
            def _():
                pltpu.make_async_copy(tkc, ekc2.at[pl.ds(0, _VB)],
                                      sem_out).wait()
                pltpu.make_async_copy(tsta, esta.at[pl.ds(0, _VB)],
                                      sem_out).wait()
                pltpu.make_async_copy(tstb, estb.at[pl.ds(0, _VB)],
                                      sem_out).wait()

            transpose_field(ekcv, 0, v0, tkc, 0)
            transpose_field(ekcv, 1, v0, tkc, _H)
            pltpu.async_copy(tkc, ekc2.at[pl.ds(v0, _VB)], sem_out)
            transpose_field(estv, 0, v0, tsta, 0)
            transpose_field(estv, 1, v0, tsta, _H)
            pltpu.async_copy(tsta, esta.at[pl.ds(v0, _VB)], sem_out)
            transpose_field(estv, 2, v0, tstb, 0)
            pltpu.async_copy(tstb, estb.at[pl.ds(v0, _VB)], sem_out)

        return carry

    lax.fori_loop(0, _K1_ITERS, block, 0)
    pltpu.make_async_copy(tkc, ekc2.at[pl.ds(0, _VB)], sem_out).wait()
    pltpu.make_async_copy(tsta, esta.at[pl.ds(0, _VB)], sem_out).wait()
    pltpu.make_async_copy(tstb, estb.at[pl.ds(0, _VB)], sem_out).wait()


@functools.partial(
    pl.kernel,
    out_type=(
        jax.ShapeDtypeStruct((_T * 6 * 8, 8, 8, 128), jnp.float32),
        jax.ShapeDtypeStruct((_T * 3 * 8, 8, 8, 128), jnp.float32),
        jax.ShapeDtypeStruct((_NST * 8, 8, 8, 128), jnp.float32),
    ),
    mesh=_MESH,
    compiler_params=_CPARAMS,
    scratch_types=[
        pltpu.VMEM((8, 128), jnp.float32),
        pltpu.VMEM((128,), jnp.int32),
        pltpu.VMEM((128,), jnp.int32),
        pltpu.VMEM((128, 2 * _H), jnp.float32),
        pltpu.VMEM((128, 2 * _H), jnp.float32),
        pltpu.VMEM((48, 8, 128), jnp.float32),
        pltpu.VMEM((24, 8, 128), jnp.float32),
        pltpu.VMEM((256,), jnp.float32),
        pltpu.VMEM((256,), jnp.float32),
        pltpu.VMEM((192,), jnp.float32),
        pltpu.VMEM((192,), jnp.float32),
        pltpu.SemaphoreType.DMA,
        pltpu.SemaphoreType.DMA,
        pltpu.SemaphoreType.DMA,
        pltpu.SemaphoreType.DMA,
        pltpu.SemaphoreType.DMA,
    ],
)
def _sc_main(scal2, kcidx, statx, ekc2, esta, estb, wkr1, bkr1, wob1, bob1,
             ko, oo, so,
             sv, i0, i1, ga, gb, kslab, oslab, wkr_v, bkr_v, wob_v, bob_v,
             sem_sv, sem_ii, sem_g, sem_ko, sem_oo):
    wid = lax.axis_index("s") * 2 + lax.axis_index("c")
    lane = lax.iota(jnp.int32, 16)
    rowbase = [lane + c * 16 for c in range(8)]

    pltpu.sync_copy(wkr1, wkr_v)
    pltpu.sync_copy(bkr1, bkr_v)
    pltpu.sync_copy(wob1, wob_v)
    pltpu.sync_copy(bob1, bob_v)

    def fire_in(u):
        t = u // 8
        b0 = pl.multiple_of((u % 8) * 128, 128)
        pltpu.async_copy(scal2.at[t, :, pl.ds(b0, 128)], sv, sem_sv)
        pltpu.async_copy(kcidx.at[t, 0, pl.ds(b0, 128)], i0, sem_ii)
        pltpu.async_copy(kcidx.at[t, 1, pl.ds(b0, 128)], i1, sem_ii)

    def dense_slab(slab, j, wref, bref, woff, row0):
        srow = [sv[j, pl.ds(c * 16, 16)] for c in range(8)]

        def hbody(h, c2):
            hq = h // 8
            hr = h % 8
            hsplat = jnp.full((16,), woff + h, jnp.int32)
            wv = plsc.load_gather(wref, [hsplat])
            bv = plsc.load_gather(bref, [hsplat])
            for c in range(8):
                slab[row0 + hq, hr, pl.ds(c * 16, 16)] = srow[c] * wv + bv
            return c2

        lax.fori_loop(0, _H, hbody, 0)

    def kc_slab(slab, g, coff, row0):
        def hbody(h, c2):
            hq = h // 8
            hr = h % 8
            colv = jnp.full((16,), coff + h, jnp.int32)
            for c in range(8):
                val = plsc.load_gather(g, [rowbase[c], colv])
                slab[row0 + hq, hr, pl.ds(c * 16, 16)] = val
            return c2

        lax.fori_loop(0, _H, hbody, 0)

    fire_in(wid * _UPW)

    def unit(ui, carry):
        u = wid * _UPW + ui
        t = u // 8
        bb = u % 8
        pltpu.make_async_copy(scal2.at[0, :, pl.ds(0, 128)], sv, sem_sv).wait()
        pltpu.make_async_copy(kcidx.at[0, 0, pl.ds(0, 128)], i0, sem_ii).wait()
        pltpu.make_async_copy(kcidx.at[0, 1, pl.ds(0, 128)], i1, sem_ii).wait()
        pltpu.async_copy(ekc2.at[i0], ga, sem_g)
        pltpu.async_copy(ekc2.at[i1], gb, sem_g)

        @pl.when(ui > 0)
        def _():
            pltpu.make_async_copy(kslab, ko.at[pl.ds(0, 48), 0], sem_ko).wait()

        for j in range(_NKR):
            dense_slab(kslab, j, wkr_v, bkr_v, j * _H, j * 8)

        @pl.when(ui > 0)
        def _():
            pltpu.make_async_copy(oslab, oo.at[pl.ds(0, 24), 0], sem_oo).wait()

        for j in range(_NOBS):
            dense_slab(oslab, _NKR + j, wob_v, bob_v, j * _H, j * 8)

        pltpu.make_async_copy(ekc2.at[pl.ds(0, 128)], ga, sem_g).wait()
        pltpu.make_async_copy(ekc2.at[pl.ds(0, 128)], gb, sem_g).wait()
        kc_slab(kslab, ga, 0, 4 * 8)
        kc_slab(kslab, gb, _H, 5 * 8)

        @pl.when(ui + 1 < _UPW)
        def _():
            fire_in(u + 1)

        pltpu.async_copy(kslab, ko.at[pl.ds(t * 48, 48), bb], sem_ko)
        pltpu.async_copy(oslab, oo.at[pl.ds(t * 24, 24), bb], sem_oo)
        return carry

    lax.fori_loop(0, _UPW, unit, 0)
    pltpu.make_async_copy(kslab, ko.at[pl.ds(0, 48), 0], sem_ko).wait()
    pltpu.make_async_copy(oslab, oo.at[pl.ds(0, 24), 0], sem_oo).wait()

    @pl.when(wid < _NST * 8)
    def _():
        f = wid // 8
        bb = wid % 8
        b0 = pl.multiple_of(bb * 128, 128)
        pltpu.sync_copy(statx.at[f, pl.ds(b0, 128)], i0)
        for ff in range(2):
            @pl.when(f == ff)
            def _():
                pltpu.async_copy(esta.at[i0], ga, sem_g).wait()

        @pl.when(f == 2)
        def _():
            pltpu.async_copy(estb.at[i0], ga, sem_g).wait()

        coff = (f % 2) * _H

        def hbody(h, c2):
            hq = h // 8
            hr = h % 8
            colv = jnp.full((16,), coff + h, jnp.int32)
            for c in range(8):
                val = plsc.load_gather(ga, [rowbase[c], colv])
                oslab[hq, hr, pl.ds(c * 16, 16)] = val
            return c2

        lax.fori_loop(0, _H, hbody, 0)
        pltpu.sync_copy(oslab.at[pl.ds(0, 8)], so.at[pl.ds(f * 8, 8), bb])


def kernel(static, known_real, known_categorical, observed,
           E_static, E_kc, W_kr, b_kr, W_obs, b_obs):
    ekcv = jnp.swapaxes(E_kc, 1, 2)
    estv = jnp.swapaxes(E_static, 1, 2)
    ekc2, esta, estb = _sc_reformat(ekcv, estv)
    scal2 = jnp.transpose(
        jnp.concatenate(
            [known_real, observed, jnp.zeros((_B, _T, 1), jnp.float32)],
            axis=-1),
        (1, 2, 0))
    kcidx = jnp.transpose(known_categorical.astype(jnp.int32), (1, 2, 0))
    statx = jnp.transpose(static.astype(jnp.int32), (1, 0))
    ko, oo, so = _sc_main(scal2, kcidx, statx, ekc2, esta, estb,
                          W_kr.reshape(-1), b_kr.reshape(-1),
                          W_obs.reshape(-1), b_obs.reshape(-1))
    known = (ko.reshape(_T, 6, 8, 8, 8, 128)
             .transpose(3, 5, 0, 2, 4, 1)
             .reshape(_B, _T, _H, 6))
    obs = (oo.reshape(_T, 3, 8, 8, 8, 128)
           .transpose(3, 5, 0, 2, 4, 1)
           .reshape(_B, _T, _H, 3))
    static_emb = (so.reshape(_NST, 8, 8, 8, 128)
                  .transpose(2, 4, 0, 1, 3)
                  .reshape(_B, _NST, _H))
    return (static_emb, known, obs)


# speedup vs baseline: 1.5122x; 1.0427x over previous
import functools

import jax
import jax.numpy as jnp
from jax import lax
from jax.experimental import pallas as pl
from jax.experimental.pallas import tpu as pltpu
from jax.experimental.pallas import tpu_sc as plsc

_B, _T, _H, _V = 1024, 200, 64, 100000
_NST, _NKC, _NKR, _NOBS = 3, 2, 4, 3
_NW = 32
_VB = 128
_VP = 100096
_NBLK = _VP // _VB
_K1_ITERS = -(-_NBLK // _NW)
_NUNIT = _T * 8
_UPW = _NUNIT // _NW

_MESH = plsc.VectorSubcoreMesh(core_axis_name="c", subcore_axis_name="s")
_CPARAMS = pltpu.CompilerParams(needs_layout_passes=False)


@functools.partial(
    pl.kernel,
    out_type=(
        jax.ShapeDtypeStruct((_VP, 2 * _H), jnp.float32),
        jax.ShapeDtypeStruct((_VP, 2 * _H), jnp.float32),
        jax.ShapeDtypeStruct((_VP, 2 * _H), jnp.float32),
    ),
    mesh=_MESH,
    compiler_params=_CPARAMS,
    scratch_types=[
        pltpu.VMEM((5, _H, _VB), jnp.float32),
        pltpu.VMEM((_VB, 2 * _H), jnp.float32),
        pltpu.VMEM((_VB, 2 * _H), jnp.float32),
        pltpu.VMEM((_VB, 2 * _H), jnp.float32),
        pltpu.SemaphoreType.DMA,
        pltpu.SemaphoreType.DMA,
        pltpu.SemaphoreType.DMA,
        pltpu.SemaphoreType.DMA,
        pltpu.SemaphoreType.DMA,
        pltpu.SemaphoreType.DMA,
    ],
)
def _sc_reformat(ekcv, estv, ekc2, esta, estb,
                 tin, tkc, tsta, tstb, si0, si1, si2, si3, si4, sem_out):
    wid = lax.axis_index("s") * 2 + lax.axis_index("c")
    lane = lax.iota(jnp.int32, 16)
    rowv = [lane + c * 16 for c in range(_VB // 16)]
    sin = [si0, si1, si2, si3, si4]

    def transpose_field(slot, dst, coff):
        pltpu.make_async_copy(ekcv.at[0, :, pl.ds(0, _VB)],
                              tin.at[slot], sin[slot]).wait()

        def hbody(h, c2):
            col = jnp.full((16,), coff + h, jnp.int32)
            for c in range(_VB // 16):
                val = tin[slot, h, pl.ds(c * 16, 16)]
                plsc.store_scatter(dst, [rowv[c], col], val)
            return c2

        lax.fori_loop(0, _H, hbody, 0)

    def block(it, carry):
        blk = it * _NW + wid

        @pl.when(blk < _NBLK)
        def _():
            v0 = pl.multiple_of(blk * _VB, _VB)
            for f in range(_NKC):
                pltpu.async_copy(ekcv.at[f, :, pl.ds(v0, _VB)],
                                 tin.at[f], sin[f])
            for f in range(_NST):
                pltpu.async_copy(estv.at[f, :, pl.ds(v0, _VB)],
                                 tin.at[_NKC + f], sin[_NKC + f])

            @pl.when(it > 0)
            def _():
                pltpu.make_async_copy(tkc, ekc2.at[pl.ds(0, _VB)],
                                      sem_out).wait()
                pltpu.make_async_copy(tsta, esta.at[pl.ds(0, _VB)],
                                      sem_out).wait()
                pltpu.make_async_copy(tstb, estb.at[pl.ds(0, _VB)],
                                      sem_out).wait()

            transpose_field(0, tkc, 0)
            transpose_field(1, tkc, _H)
            pltpu.async_copy(tkc, ekc2.at[pl.ds(v0, _VB)], sem_out)
            transpose_field(2, tsta, 0)
            transpose_field(3, tsta, _H)
            pltpu.async_copy(tsta, esta.at[pl.ds(v0, _VB)], sem_out)
            transpose_field(4, tstb, 0)
            pltpu.async_copy(tstb, estb.at[pl.ds(v0, _VB)], sem_out)

        return carry

    lax.fori_loop(0, _K1_ITERS, block, 0)
    pltpu.make_async_copy(tkc, ekc2.at[pl.ds(0, _VB)], sem_out).wait()
    pltpu.make_async_copy(tsta, esta.at[pl.ds(0, _VB)], sem_out).wait()
    pltpu.make_async_copy(tstb, estb.at[pl.ds(0, _VB)], sem_out).wait()


@functools.partial(
    pl.kernel,
    out_type=(
        jax.ShapeDtypeStruct((_T * 6 * 8, 8, 8, 128), jnp.float32),
        jax.ShapeDtypeStruct((_T * 3 * 8, 8, 8, 128), jnp.float32),
        jax.ShapeDtypeStruct((_NST * 8, 8, 8, 128), jnp.float32),
    ),
    mesh=_MESH,
    compiler_params=_CPARAMS,
    scratch_types=[
        pltpu.VMEM((8, 128), jnp.float32),
        pltpu.VMEM((128,), jnp.int32),
        pltpu.VMEM((128,), jnp.int32),
        pltpu.VMEM((128, 2 * _H), jnp.float32),
        pltpu.VMEM((128, 2 * _H), jnp.float32),
        pltpu.VMEM((48, 8, 128), jnp.float32),
        pltpu.VMEM((24, 8, 128), jnp.float32),
        pltpu.VMEM((256,), jnp.float32),
        pltpu.VMEM((256,), jnp.float32),
        pltpu.VMEM((192,), jnp.float32),
        pltpu.VMEM((192,), jnp.float32),
        pltpu.SemaphoreType.DMA,
        pltpu.SemaphoreType.DMA,
        pltpu.SemaphoreType.DMA,
        pltpu.SemaphoreType.DMA,
        pltpu.SemaphoreType.DMA,
    ],
)
def _sc_main(scal2, kcidx, statx, ekc2, esta, estb, wkr1, bkr1, wob1, bob1,
             ko, oo, so,
             sv, i0, i1, ga, gb, kslab, oslab, wkr_v, bkr_v, wob_v, bob_v,
             sem_sv, sem_ii, sem_g, sem_ko, sem_oo):
    wid = lax.axis_index("s") * 2 + lax.axis_index("c")
    lane = lax.iota(jnp.int32, 16)
    rowbase = [lane + c * 16 for c in range(8)]

    pltpu.sync_copy(wkr1, wkr_v)
    pltpu.sync_copy(bkr1, bkr_v)
    pltpu.sync_copy(wob1, wob_v)
    pltpu.sync_copy(bob1, bob_v)

    def fire_in(u):
        t = u // 8
        b0 = pl.multiple_of((u % 8) * 128, 128)
        pltpu.async_copy(scal2.at[t, :, pl.ds(b0, 128)], sv, sem_sv)
        pltpu.async_copy(kcidx.at[t, 0, pl.ds(b0, 128)], i0, sem_ii)
        pltpu.async_copy(kcidx.at[t, 1, pl.ds(b0, 128)], i1, sem_ii)

    def dense_slab(slab, j, wref, bref, woff, row0):
        srow = [sv[j, pl.ds(c * 16, 16)] for c in range(8)]

        def hbody(h, c2):
            hq = h // 8
            hr = h % 8
            hsplat = jnp.full((16,), woff + h, jnp.int32)
            wv = plsc.load_gather(wref, [hsplat])
            bv = plsc.load_gather(bref, [hsplat])
            for c in range(8):
                slab[row0 + hq, hr, pl.ds(c * 16, 16)] = srow[c] * wv + bv
            return c2

        lax.fori_loop(0, _H, hbody, 0)

    def kc_slab(slab, g, coff, row0):
        def hbody(h, c2):
            hq = h // 8
            hr = h % 8
            colv = jnp.full((16,), coff + h, jnp.int32)
            for c in range(8):
                val = plsc.load_gather(g, [rowbase[c], colv])
                slab[row0 + hq, hr, pl.ds(c * 16, 16)] = val
            return c2

        lax.fori_loop(0, _H, hbody, 0)

    fire_in(wid * _UPW)

    def unit(ui, carry):
        u = wid * _UPW + ui
        t = u // 8
        bb = u % 8
        pltpu.make_async_copy(scal2.at[0, :, pl.ds(0, 128)], sv, sem_sv).wait()
        pltpu.make_async_copy(kcidx.at[0, 0, pl.ds(0, 128)], i0, sem_ii).wait()
        pltpu.make_async_copy(kcidx.at[0, 1, pl.ds(0, 128)], i1, sem_ii).wait()
        pltpu.async_copy(ekc2.at[i0], ga, sem_g)
        pltpu.async_copy(ekc2.at[i1], gb, sem_g)

        @pl.when(ui > 0)
        def _():
            pltpu.make_async_copy(kslab, ko.at[pl.ds(0, 48), 0], sem_ko).wait()

        for j in range(_NKR):
            dense_slab(kslab, j, wkr_v, bkr_v, j * _H, j * 8)

        @pl.when(ui > 0)
        def _():
            pltpu.make_async_copy(oslab, oo.at[pl.ds(0, 24), 0], sem_oo).wait()

        for j in range(_NOBS):
            dense_slab(oslab, _NKR + j, wob_v, bob_v, j * _H, j * 8)

        pltpu.make_async_copy(ekc2.at[pl.ds(0, 128)], ga, sem_g).wait()
        pltpu.make_async_copy(ekc2.at[pl.ds(0, 128)], gb, sem_g).wait()
        kc_slab(kslab, ga, 0, 4 * 8)
        kc_slab(kslab, gb, _H, 5 * 8)

        @pl.when(ui + 1 < _UPW)
        def _():
            fire_in(u + 1)

        pltpu.async_copy(kslab, ko.at[pl.ds(t * 48, 48), bb], sem_ko)
        pltpu.async_copy(oslab, oo.at[pl.ds(t * 24, 24), bb], sem_oo)
        return carry

    lax.fori_loop(0, _UPW, unit, 0)
    pltpu.make_async_copy(kslab, ko.at[pl.ds(0, 48), 0], sem_ko).wait()
    pltpu.make_async_copy(oslab, oo.at[pl.ds(0, 24), 0], sem_oo).wait()

    @pl.when(wid < _NST * 8)
    def _():
        f = wid // 8
        bb = wid % 8
        b0 = pl.multiple_of(bb * 128, 128)
        pltpu.sync_copy(statx.at[f, pl.ds(b0, 128)], i0)
        for ff in range(2):
            @pl.when(f == ff)
            def _():
                pltpu.async_copy(esta.at[i0], ga, sem_g).wait()

        @pl.when(f == 2)
        def _():
            pltpu.async_copy(estb.at[i0], ga, sem_g).wait()

        coff = (f % 2) * _H

        def hbody(h, c2):
            hq = h // 8
            hr = h % 8
            colv = jnp.full((16,), coff + h, jnp.int32)
            for c in range(8):
                val = plsc.load_gather(ga, [rowbase[c], colv])
                oslab[hq, hr, pl.ds(c * 16, 16)] = val
            return c2

        lax.fori_loop(0, _H, hbody, 0)
        pltpu.sync_copy(oslab.at[pl.ds(0, 8)], so.at[pl.ds(f * 8, 8), bb])


def kernel(static, known_real, known_categorical, observed,
           E_static, E_kc, W_kr, b_kr, W_obs, b_obs):
    ekcv = jnp.swapaxes(E_kc, 1, 2)
    estv = jnp.swapaxes(E_static, 1, 2)
    ekc2, esta, estb = _sc_reformat(ekcv, estv)
    scal2 = jnp.transpose(
        jnp.concatenate(
            [known_real, observed, jnp.zeros((_B, _T, 1), jnp.float32)],
            axis=-1),
        (1, 2, 0))
    kcidx = jnp.transpose(known_categorical.astype(jnp.int32), (1, 2, 0))
    statx = jnp.transpose(static.astype(jnp.int32), (1, 0))
    ko, oo, so = _sc_main(scal2, kcidx, statx, ekc2, esta, estb,
                          W_kr.reshape(-1), b_kr.reshape(-1),
                          W_obs.reshape(-1), b_obs.reshape(-1))
    known = (ko.reshape(_T, 6, 8, 8, 8, 128)
             .transpose(3, 5, 0, 2, 4, 1)
             .reshape(_B, _T, _H, 6))
    obs = (oo.reshape(_T, 3, 8, 8, 8, 128)
           .transpose(3, 5, 0, 2, 4, 1)
           .reshape(_B, _T, _H, 3))
    static_emb = (so.reshape(_NST, 8, 8, 8, 128)
                  .transpose(2, 4, 0, 1, 3)
                  .reshape(_B, _NST, _H))
    return (static_emb, known, obs)


# speedup vs baseline: 1.5361x; 1.0158x over previous
import functools

import jax
import jax.numpy as jnp
from jax import lax
from jax.experimental import pallas as pl
from jax.experimental.pallas import tpu as pltpu
from jax.experimental.pallas import tpu_sc as plsc

_B, _T, _H, _V = 1024, 200, 64, 100000
_NST, _NKC, _NKR, _NOBS = 3, 2, 4, 3
_NW = 32
_VB = 128
_VP = 100096
_NBLK = _VP // _VB
_K1_ITERS = -(-_NBLK // _NW)
_NUNIT = _T * 8
_UPW = _NUNIT // _NW

_MESH = plsc.VectorSubcoreMesh(core_axis_name="c", subcore_axis_name="s")
_CPARAMS = pltpu.CompilerParams(needs_layout_passes=False)


@functools.partial(
    pl.kernel,
    out_type=(
        jax.ShapeDtypeStruct((_VP, 2 * _H), jnp.float32),
        jax.ShapeDtypeStruct((_VP, 2 * _H), jnp.float32),
        jax.ShapeDtypeStruct((_VP, 2 * _H), jnp.float32),
    ),
    mesh=_MESH,
    compiler_params=_CPARAMS,
    scratch_types=[
        pltpu.VMEM((5, _H, _VB), jnp.float32),
        pltpu.VMEM((_VB, 2 * _H), jnp.float32),
        pltpu.VMEM((_VB, 2 * _H), jnp.float32),
        pltpu.VMEM((_VB, 2 * _H), jnp.float32),
        pltpu.SemaphoreType.DMA,
        pltpu.SemaphoreType.DMA,
        pltpu.SemaphoreType.DMA,
        pltpu.SemaphoreType.DMA,
        pltpu.SemaphoreType.DMA,
        pltpu.SemaphoreType.DMA,
    ],
)
def _sc_reformat(ekcv, estv, ekc2, esta, estb,
                 tin, tkc, tsta, tstb, si0, si1, si2, si3, si4, sem_out):
    wid = lax.axis_index("s") * 2 + lax.axis_index("c")
    lane = lax.iota(jnp.int32, 16)
    rowv = [lane + c * 16 for c in range(_VB // 16)]
    sin = [si0, si1, si2, si3, si4]

    def transpose_field(slot, dst, coff):
        pltpu.make_async_copy(ekcv.at[0, :, pl.ds(0, _VB)],
                              tin.at[slot], sin[slot]).wait()

        def hbody(h, c2):
            col = jnp.full((16,), coff + h, jnp.int32)
            for c in range(_VB // 16):
                val = tin[slot, h, pl.ds(c * 16, 16)]
                plsc.store_scatter(dst, [rowv[c], col], val)
            return c2

        lax.fori_loop(0, _H, hbody, 0)

    def block(it, carry):
        blk = it * _NW + wid

        @pl.when(blk < _NBLK)
        def _():
            v0 = pl.multiple_of(blk * _VB, _VB)
            for f in range(_NKC):
                pltpu.async_copy(ekcv.at[f, :, pl.ds(v0, _VB)],
                                 tin.at[f], sin[f])
            for f in range(_NST):
                pltpu.async_copy(estv.at[f, :, pl.ds(v0, _VB)],
                                 tin.at[_NKC + f], sin[_NKC + f])

            @pl.when(it > 0)
            def _():
                pltpu.make_async_copy(tkc, ekc2.at[pl.ds(0, _VB)],
                                      sem_out).wait()
                pltpu.make_async_copy(tsta, esta.at[pl.ds(0, _VB)],
                                      sem_out).wait()
                pltpu.make_async_copy(tstb, estb.at[pl.ds(0, _VB)],
                                      sem_out).wait()

            transpose_field(0, tkc, 0)
            transpose_field(1, tkc, _H)
            pltpu.async_copy(tkc, ekc2.at[pl.ds(v0, _VB)], sem_out)
            transpose_field(2, tsta, 0)
            transpose_field(3, tsta, _H)
            pltpu.async_copy(tsta, esta.at[pl.ds(v0, _VB)], sem_out)
            transpose_field(4, tstb, 0)
            pltpu.async_copy(tstb, estb.at[pl.ds(v0, _VB)], sem_out)

        return carry

    lax.fori_loop(0, _K1_ITERS, block, 0)
    pltpu.make_async_copy(tkc, ekc2.at[pl.ds(0, _VB)], sem_out).wait()
    pltpu.make_async_copy(tsta, esta.at[pl.ds(0, _VB)], sem_out).wait()
    pltpu.make_async_copy(tstb, estb.at[pl.ds(0, _VB)], sem_out).wait()


@functools.partial(
    pl.kernel,
    out_type=(
        jax.ShapeDtypeStruct((_T * 6 * 8, 8, 8, 128), jnp.float32),
        jax.ShapeDtypeStruct((_T * 3 * 8, 8, 8, 128), jnp.float32),
        jax.ShapeDtypeStruct((_NST * 8, 8, 8, 128), jnp.float32),
    ),
    mesh=_MESH,
    compiler_params=_CPARAMS,
    scratch_types=[
        pltpu.VMEM((8, 128), jnp.float32),
        pltpu.VMEM((128,), jnp.int32),
        pltpu.VMEM((128,), jnp.int32),
        pltpu.VMEM((128, 2 * _H), jnp.float32),
        pltpu.VMEM((128, 2 * _H), jnp.float32),
        pltpu.VMEM((48, 8, 128), jnp.float32),
        pltpu.VMEM((24, 8, 128), jnp.float32),
        pltpu.VMEM((256,), jnp.float32),
        pltpu.VMEM((256,), jnp.float32),
        pltpu.VMEM((192,), jnp.float32),
        pltpu.VMEM((192,), jnp.float32),
        pltpu.SemaphoreType.DMA,
        pltpu.SemaphoreType.DMA,
        pltpu.SemaphoreType.DMA,
        pltpu.SemaphoreType.DMA,
        pltpu.SemaphoreType.DMA,
    ],
)
def _sc_main(scal2, kcidx, statx, ekc2, esta, estb, wkr1, bkr1, wob1, bob1,
             ko, oo, so,
             sv, i0, i1, ga, gb, kslab, oslab, wkr_v, bkr_v, wob_v, bob_v,
             sem_sv, sem_ii, sem_g, sem_ko, sem_oo):
    wid = lax.axis_index("s") * 2 + lax.axis_index("c")
    lane = lax.iota(jnp.int32, 16)
    rowbase = [lane + c * 16 for c in range(8)]

    pltpu.sync_copy(wkr1, wkr_v)
    pltpu.sync_copy(bkr1, bkr_v)
    pltpu.sync_copy(wob1, wob_v)
    pltpu.sync_copy(bob1, bob_v)

    def fire_in(u):
        t = u // 8
        b0 = pl.multiple_of((u % 8) * 128, 128)
        pltpu.async_copy(scal2.at[t, :, pl.ds(b0, 128)], sv, sem_sv)
        pltpu.async_copy(kcidx.at[t, 0, pl.ds(b0, 128)], i0, sem_ii)
        pltpu.async_copy(kcidx.at[t, 1, pl.ds(b0, 128)], i1, sem_ii)

    def dense_slab(slab, j, wref, bref, woff, row0):
        srow = [sv[j, pl.ds(c * 16, 16)] for c in range(8)]

        def hbody(h, c2):
            hq = h // 8
            hr = h % 8
            hsplat = jnp.full((16,), woff + h, jnp.int32)
            wv = plsc.load_gather(wref, [hsplat])
            for c in range(8):
                slab[row0 + hq, hr, pl.ds(c * 16, 16)] = srow[c] * wv
            return c2

        lax.fori_loop(0, _H, hbody, 0)

    def kc_slab(slab, g, coff, row0):
        def hbody(h, c2):
            hq = h // 8
            hr = h % 8
            colv = jnp.full((16,), coff + h, jnp.int32)
            for c in range(8):
                val = plsc.load_gather(g, [rowbase[c], colv])
                slab[row0 + hq, hr, pl.ds(c * 16, 16)] = val
            return c2

        lax.fori_loop(0, _H, hbody, 0)

    fire_in(wid * _UPW)

    def unit(ui, carry):
        u = wid * _UPW + ui
        t = u // 8
        bb = u % 8
        pltpu.make_async_copy(scal2.at[0, :, pl.ds(0, 128)], sv, sem_sv).wait()
        pltpu.make_async_copy(kcidx.at[0, 0, pl.ds(0, 128)], i0, sem_ii).wait()
        pltpu.make_async_copy(kcidx.at[0, 1, pl.ds(0, 128)], i1, sem_ii).wait()
        pltpu.async_copy(ekc2.at[i0], ga, sem_g)
        pltpu.async_copy(ekc2.at[i1], gb, sem_g)

        @pl.when(ui > 0)
        def _():
            pltpu.make_async_copy(kslab, ko.at[pl.ds(0, 48), 0], sem_ko).wait()

        for j in range(_NKR):
            dense_slab(kslab, j, wkr_v, bkr_v, j * _H, j * 8)

        @pl.when(ui > 0)
        def _():
            pltpu.make_async_copy(oslab, oo.at[pl.ds(0, 24), 0], sem_oo).wait()

        for j in range(_NOBS):
            dense_slab(oslab, _NKR + j, wob_v, bob_v, j * _H, j * 8)

        pltpu.make_async_copy(ekc2.at[pl.ds(0, 128)], ga, sem_g).wait()
        pltpu.make_async_copy(ekc2.at[pl.ds(0, 128)], gb, sem_g).wait()
        kc_slab(kslab, ga, 0, 4 * 8)
        kc_slab(kslab, gb, _H, 5 * 8)

        @pl.when(ui + 1 < _UPW)
        def _():
            fire_in(u + 1)

        pltpu.async_copy(kslab, ko.at[pl.ds(t * 48, 48), bb], sem_ko)
        pltpu.async_copy(oslab, oo.at[pl.ds(t * 24, 24), bb], sem_oo)
        return carry

    lax.fori_loop(0, _UPW, unit, 0)
    pltpu.make_async_copy(kslab, ko.at[pl.ds(0, 48), 0], sem_ko).wait()
    pltpu.make_async_copy(oslab, oo.at[pl.ds(0, 24), 0], sem_oo).wait()

    @pl.when(wid < _NST * 8)
    def _():
        f = wid // 8
        bb = wid % 8
        b0 = pl.multiple_of(bb * 128, 128)
        pltpu.sync_copy(statx.at[f, pl.ds(b0, 128)], i0)
        for ff in range(2):
            @pl.when(f == ff)
            def _():
                pltpu.async_copy(esta.at[i0], ga, sem_g).wait()

        @pl.when(f == 2)
        def _():
            pltpu.async_copy(estb.at[i0], ga, sem_g).wait()

        coff = (f % 2) * _H

        def hbody(h, c2):
            hq = h // 8
            hr = h % 8
            colv = jnp.full((16,), coff + h, jnp.int32)
            for c in range(8):
                val = plsc.load_gather(ga, [rowbase[c], colv])
                oslab[hq, hr, pl.ds(c * 16, 16)] = val
            return c2

        lax.fori_loop(0, _H, hbody, 0)
        pltpu.sync_copy(oslab.at[pl.ds(0, 8)], so.at[pl.ds(f * 8, 8), bb])


def kernel(static, known_real, known_categorical, observed,
           E_static, E_kc, W_kr, b_kr, W_obs, b_obs):
    ekcv = jnp.swapaxes(E_kc, 1, 2)
    estv = jnp.swapaxes(E_static, 1, 2)
    ekc2, esta, estb = _sc_reformat(ekcv, estv)
    scal2 = jnp.transpose(
        jnp.concatenate(
            [known_real, observed, jnp.zeros((_B, _T, 1), jnp.float32)],
            axis=-1),
        (1, 2, 0))
    kcidx = jnp.transpose(known_categorical.astype(jnp.int32), (1, 2, 0))
    statx = jnp.transpose(static.astype(jnp.int32), (1, 0))
    ko, oo, so = _sc_main(scal2, kcidx, statx, ekc2, esta, estb,
                          W_kr.reshape(-1), b_kr.reshape(-1),
                          W_obs.reshape(-1), b_obs.reshape(-1))
    known = (ko.reshape(_T, 6, 8, 8, 8, 128)
             .transpose(3, 5, 0, 2, 4, 1)
             .reshape(_B, _T, _H, 6))
    obs = (oo.reshape(_T, 3, 8, 8, 8, 128)
           .transpose(3, 5, 0, 2, 4, 1)
           .reshape(_B, _T, _H, 3))
    static_emb = (so.reshape(_NST, 8, 8, 8, 128)
                  .transpose(2, 4, 0, 1, 3)
                  .reshape(_B, _NST, _H))
    return (static_emb, known, obs)
